# baseline (speedup 1.0000x reference)
import functools

import jax
import jax.numpy as jnp
from jax import lax
from jax.experimental import pallas as pl
from jax.experimental.pallas import tpu as pltpu
from jax.experimental.pallas import tpu_sc as plsc

_BATCH = 16384
_HIST = 20
_DIM = 64
_B = _BATCH * _HIST

_info = plsc.get_sparse_core_info()
_NC, _NS = _info.num_cores, _info.num_subcores
_NW = _NC * _NS
_B_PER_W = _B // _NW
_CHUNK = 1024
_N_CHUNKS = _B_PER_W // _CHUNK


@functools.partial(
    pl.kernel,
    mesh=plsc.VectorSubcoreMesh(core_axis_name="c", subcore_axis_name="s"),
    out_type=jax.ShapeDtypeStruct((_B, _DIM), jnp.float32),
    compiler_params=pltpu.CompilerParams(use_tc_tiling_on_sc=False),
    scratch_types=[
        pltpu.VMEM((_CHUNK,), jnp.int32),
        pltpu.VMEM((_CHUNK, _DIM), jnp.float32),
        pltpu.SemaphoreType.DMA,
    ],
)
def _gather_rows(x_hbm, table_hbm, out_hbm, idx_v, rows_v, sem):
    wid = lax.axis_index("s") * _NC + lax.axis_index("c")
    base = wid * _B_PER_W

    def chunk_body(i, carry):
        off = base + i * _CHUNK
        pltpu.sync_copy(x_hbm.at[pl.ds(off, _CHUNK)], idx_v)
        pltpu.async_copy(table_hbm.at[idx_v], rows_v, sem).wait()
        pltpu.sync_copy(rows_v, out_hbm.at[pl.ds(off, _CHUNK)])
        return carry

    lax.fori_loop(0, _N_CHUNKS, chunk_body, 0)


def kernel(x, table):
    flat = _gather_rows(x.reshape(_B), table)
    return flat.reshape(_BATCH, _HIST, _DIM)


# speedup vs baseline: 1.0059x; 1.0059x over previous
import functools

import jax
import jax.numpy as jnp
from jax import lax
from jax.experimental import pallas as pl
from jax.experimental.pallas import tpu as pltpu
from jax.experimental.pallas import tpu_sc as plsc

_BATCH = 16384
_HIST = 20
_DIM = 64
_B = _BATCH * _HIST

_info = plsc.get_sparse_core_info()
_NC, _NS = _info.num_cores, _info.num_subcores
_NW = _NC * _NS
_B_PER_W = _B // _NW
_CHUNK = 512
_N_CHUNKS = _B_PER_W // _CHUNK
_NBUF = 3


@functools.partial(
    pl.kernel,
    mesh=plsc.VectorSubcoreMesh(core_axis_name="c", subcore_axis_name="s"),
    out_type=jax.ShapeDtypeStruct((_B, _DIM), jnp.float32),
    compiler_params=pltpu.CompilerParams(use_tc_tiling_on_sc=False),
    scratch_types=[
        pltpu.VMEM((_N_CHUNKS, _CHUNK), jnp.int32),
        pltpu.VMEM((_NBUF, _CHUNK, _DIM), jnp.float32),
        pltpu.SemaphoreType.DMA((_NBUF,)),
        pltpu.SemaphoreType.DMA((_NBUF,)),
    ],
)
def _gather_rows(x_hbm, table_hbm, out_hbm, idx_all, rows, sem_g, sem_o):
    wid = lax.axis_index("s") * _NC + lax.axis_index("c")
    cbase = wid * _N_CHUNKS
    pltpu.sync_copy(x_hbm.at[pl.ds(cbase, _N_CHUNKS)], idx_all)

    def gather(c):
        b = c % _NBUF
        return pltpu.async_copy(
            table_hbm.at[idx_all.at[c]], rows.at[b], sem_g.at[b]
        )

    def writeout(c):
        b = c % _NBUF
        return pltpu.async_copy(
            rows.at[b],
            out_hbm.at[pl.ds((cbase + c) * _CHUNK, _CHUNK)],
            sem_o.at[b],
        )

    cp_g = [None] * _N_CHUNKS
    cp_o = [None] * _N_CHUNKS
    for b in range(_NBUF):
        cp_g[b] = gather(b)
    for c in range(_N_CHUNKS):
        if c > 0:
            cp_o[c - 1].wait()
            nxt = c - 1 + _NBUF
            if nxt < _N_CHUNKS:
                cp_g[nxt] = gather(nxt)
        cp_g[c].wait()
        cp_o[c] = writeout(c)
    cp_o[_N_CHUNKS - 1].wait()


def kernel(x, table):
    flat = _gather_rows(x.reshape(_NW * _N_CHUNKS, _CHUNK), table)
    return flat.reshape(_BATCH, _HIST, _DIM)


# speedup vs baseline: 1.0098x; 1.0039x over previous
import functools

import jax
import jax.numpy as jnp
from jax import lax
from jax.experimental import pallas as pl
from jax.experimental.pallas import tpu as pltpu
from jax.experimental.pallas import tpu_sc as plsc

_BATCH = 16384
_HIST = 20
_DIM = 64

_info = plsc.get_sparse_core_info()
_NC, _NS = _info.num_cores, _info.num_subcores
_NW = _NC * _NS
_BPW = _BATCH // _NW
_NBUF = 3


@functools.partial(
    pl.kernel,
    mesh=plsc.VectorSubcoreMesh(core_axis_name="c", subcore_axis_name="s"),
    out_type=jax.ShapeDtypeStruct((_BATCH, _HIST, _DIM), jnp.float32),
    compiler_params=pltpu.CompilerParams(use_tc_tiling_on_sc=False),
    scratch_types=[
        pltpu.VMEM((_HIST, _BPW), jnp.int32),
        pltpu.VMEM((_NBUF, _BPW, _DIM), jnp.float32),
        pltpu.SemaphoreType.DMA((_NBUF,)),
        pltpu.SemaphoreType.DMA((_NBUF,)),
    ],
)
def _gather_rows(xt_hbm, table_hbm, out_hbm, idx_v, rows, sem_g, sem_o):
    wid = lax.axis_index("s") * _NC + lax.axis_index("c")
    b0 = wid * _BPW
    pltpu.sync_copy(xt_hbm.at[:, pl.ds(b0, _BPW)], idx_v)

    def gather(h):
        b = h % _NBUF
        return pltpu.async_copy(
            table_hbm.at[idx_v.at[h]], rows.at[b], sem_g.at[b]
        )

    def writeout(h):
        b = h % _NBUF
        return pltpu.async_copy(
            rows.at[b],
            out_hbm.at[pl.ds(b0, _BPW), h],
            sem_o.at[b],
        )

    cp_g = [None] * _HIST
    cp_o = [None] * _HIST
    for b in range(_NBUF):
        cp_g[b] = gather(b)
    for h in range(_HIST):
        if h > 0:
            cp_o[h - 1].wait()
            nxt = h - 1 + _NBUF
            if nxt < _HIST:
                cp_g[nxt] = gather(nxt)
        cp_g[h].wait()
        cp_o[h] = writeout(h)
    cp_o[_HIST - 1].wait()


def kernel(x, table):
    return _gather_rows(x.T, table)
